# Initial kernel scaffold; baseline (speedup 1.0000x reference)
import jax
import jax.numpy as jnp
from jax.experimental import pallas as pl


def kernel(x, edge_index, W1a, b1a, W1b, b1b, W2a, b2a, W2b, b2b, Q):
    raise NotImplementedError("write your pallas kernel here")


# speedup vs baseline: 3.4484x; 3.4484x over previous
import functools

import jax
import jax.numpy as jnp
from jax import lax
from jax.experimental import pallas as pl
from jax.experimental.pallas import tpu as pltpu
from jax.experimental.pallas import tpu_sc as plsc

N = 10000
E = 320000
D = 128
C = 40
EPS = 0.0

NC = 2
NS = 16
NW = NC * NS

K = 128
CHUNKS = 79
EPW = K * CHUNKS
E_PAD = EPW * NW
NPAD = 10240
ROWS_PER_TILE = NPAD // NS


def _seg_sum_body(h_hbm, src_hbm, dst_hbm, out_hbm,
                  acc_sh, srcv, dstv, rows, zblk, sem):
    c = lax.axis_index("c")
    s = lax.axis_index("s")
    w = s * NC + c

    zero = jnp.zeros((16,), jnp.float32)
    for r in range(16):
        for b in range(8):
            zblk[r, pl.ds(b * 16, 16)] = zero

    @pl.loop(0, ROWS_PER_TILE // 16)
    def _zero(j):
        pltpu.sync_copy(zblk, acc_sh.at[pl.ds(s * ROWS_PER_TILE + j * 16, 16)])

    plsc.subcore_barrier()

    @pl.loop(0, CHUNKS)
    def _edges(i):
        base = w * EPW + i * K
        pltpu.sync_copy(src_hbm.at[pl.ds(base, K)], srcv)
        pltpu.sync_copy(dst_hbm.at[pl.ds(base, K)], dstv)
        pltpu.async_copy(h_hbm.at[srcv], rows, sem).wait()
        pltpu.sync_copy(rows, acc_sh.at[dstv], add=True)

    plsc.subcore_barrier()

    pltpu.sync_copy(acc_sh.at[pl.ds(s * ROWS_PER_TILE, ROWS_PER_TILE)],
                    out_hbm.at[c, pl.ds(s * ROWS_PER_TILE, ROWS_PER_TILE)])


def _segment_sum(h, src_pad, dst_pad):
    mesh = plsc.VectorSubcoreMesh(core_axis_name="c", subcore_axis_name="s")
    return pl.kernel(
        _seg_sum_body,
        out_type=jax.ShapeDtypeStruct((NC, NPAD, D), jnp.float32),
        mesh=mesh,
        scratch_types=[
            pltpu.VMEM_SHARED((NPAD, D), jnp.float32),
            pltpu.VMEM((K,), jnp.int32),
            pltpu.VMEM((K,), jnp.int32),
            pltpu.VMEM((K, D), jnp.float32),
            pltpu.VMEM((16, D), jnp.float32),
            pltpu.SemaphoreType.DMA,
        ],
    )(h, src_pad, dst_pad)


def _mlp1_body(x_ref, p_ref, wa_ref, ba_ref, wb_ref, bb_ref, o_ref):
    a = x_ref[...] * (1.0 + EPS) + p_ref[0] + p_ref[1]
    t = jnp.maximum(
        jnp.dot(a, wa_ref[...], preferred_element_type=jnp.float32) + ba_ref[...],
        0.0)
    u = jnp.dot(t, wb_ref[...], preferred_element_type=jnp.float32) + bb_ref[...]
    o_ref[...] = jnp.where(u > 0.0, u, jnp.exp(u) - 1.0)


def _mlp2_body(x_ref, p_ref, wa_ref, ba_ref, wb_ref, bb_ref, o_ref):
    a = x_ref[...] * (1.0 + EPS) + p_ref[0] + p_ref[1]
    t = jnp.maximum(
        jnp.dot(a, wa_ref[...], preferred_element_type=jnp.float32) + ba_ref[...],
        0.0)
    o_ref[...] = jnp.dot(t, wb_ref[...], preferred_element_type=jnp.float32) + bb_ref[...]


def _mlp(body, x, parts, Wa, ba, Wb, bb, out_cols):
    R = 1000
    grid = N // R
    return pl.pallas_call(
        body,
        grid=(grid,),
        in_specs=[
            pl.BlockSpec((R, D), lambda i: (i, 0)),
            pl.BlockSpec((NC, R, D), lambda i: (0, i, 0)),
            pl.BlockSpec(Wa.shape, lambda i: (0, 0)),
            pl.BlockSpec(ba.shape, lambda i: (0, 0)),
            pl.BlockSpec(Wb.shape, lambda i: (0, 0)),
            pl.BlockSpec(bb.shape, lambda i: (0, 0)),
        ],
        out_specs=pl.BlockSpec((R, out_cols), lambda i: (i, 0)),
        out_shape=jax.ShapeDtypeStruct((N, out_cols), jnp.float32),
    )(x, parts, Wa, ba, Wb, bb)


def kernel(x, edge_index, W1a, b1a, W1b, b1b, W2a, b2a, W2b, b2b, Q):
    src = edge_index[0].astype(jnp.int32)
    dst = edge_index[1].astype(jnp.int32)
    pad = E_PAD - E
    src_pad = jnp.concatenate([src, jnp.zeros((pad,), jnp.int32)])
    dst_pad = jnp.concatenate([dst, jnp.full((pad,), N, jnp.int32)])

    W2b_p = jnp.zeros((D, D), jnp.float32).at[:, :C].set(W2b)
    b2b_p = jnp.zeros((1, D), jnp.float32).at[0, :C].set(b2b)

    agg1 = _segment_sum(x, src_pad, dst_pad)
    h = _mlp(_mlp1_body, x, agg1, W1a, b1a.reshape(1, D), W1b,
             b1b.reshape(1, D), D)
    agg2 = _segment_sum(h, src_pad, dst_pad)
    out = _mlp(_mlp2_body, h, agg2, W2a, b2a.reshape(1, D), W2b_p, b2b_p, D)
    return (out[:, :C], Q)
